# Initial kernel scaffold; baseline (speedup 1.0000x reference)
import jax
import jax.numpy as jnp
from jax.experimental import pallas as pl


def kernel(x, edge_index, edge_attr, params):
    raise NotImplementedError("write your pallas kernel here")


# speedup vs baseline: 1.6476x; 1.6476x over previous
import functools

import numpy as np
import jax
import jax.numpy as jnp
from jax import lax
from jax.experimental import pallas as pl
from jax.experimental.pallas import tpu as pltpu
from jax.experimental.pallas import tpu_sc as plsc

N = 10000
E = 320000
D = 128
ED = 16

NRANGES = 64
RSIZE = 160
NPAD = NRANGES * RSIZE
TRASH = RSIZE
STAGE = 512
CHUNK = 32
FLUSH = 1024
BUF = 1536
ECAP = E + 2048
PSTAGE = 10000

_HIST = np.concatenate([np.zeros(32), np.array([10000.0])])
_BINS = np.arange(_HIST.shape[0], dtype=np.float64)
_AVG_LOG = float((np.log(_BINS + 1.0) * _HIST).sum() / _HIST.sum())

_SC_MESH = dict(core_axis_name="c", subcore_axis_name="s")


def _make_partition():
    mesh = plsc.VectorSubcoreMesh(**_SC_MESH)
    out_type = (
        jax.ShapeDtypeStruct((NRANGES * ECAP,), jnp.int32),
        jax.ShapeDtypeStruct((NRANGES * ECAP,), jnp.int32),
        jax.ShapeDtypeStruct((NRANGES * ECAP,), jnp.int32),
        jax.ShapeDtypeStruct((NRANGES * 16,), jnp.int32),
    )
    scratch = [
        pltpu.VMEM((PSTAGE,), jnp.int32),
        pltpu.VMEM((PSTAGE,), jnp.int32),
        pltpu.VMEM((PSTAGE,), jnp.int32),
        pltpu.VMEM((PSTAGE,), jnp.int32),
        pltpu.VMEM((BUF,), jnp.int32),
        pltpu.VMEM((BUF,), jnp.int32),
        pltpu.VMEM((BUF,), jnp.int32),
        pltpu.VMEM((BUF,), jnp.int32),
        pltpu.VMEM((BUF,), jnp.int32),
        pltpu.VMEM((BUF,), jnp.int32),
        pltpu.VMEM((16,), jnp.int32),
        pltpu.SemaphoreType.DMA,
        pltpu.SemaphoreType.DMA,
    ]

    @functools.partial(
        pl.kernel, out_type=out_type, mesh=mesh,
        compiler_params=pltpu.CompilerParams(needs_layout_passes=False),
        scratch_types=scratch)
    def part(dst_hbm, src_hbm, srcg_hbm, dstg_hbm, idsg_hbm, cnt_hbm,
             dstA, srcA, dstB, srcB,
             b0s, b0d, b0i, b1s, b1d, b1i, cbuf, semA, semB):
        wid = lax.axis_index("s") * 2 + lax.axis_index("c")
        r0 = wid * 2
        r1 = r0 + 1
        base0 = r0 * RSIZE
        base1 = r1 * RSIZE
        lane = lax.iota(jnp.int32, 16)

        def issue_stage(si, dref, sref, sem):
            off = pl.multiple_of(si * PSTAGE, 16)
            pltpu.make_async_copy(
                dst_hbm.at[pl.ds(off, PSTAGE)], dref, sem).start()
            pltpu.make_async_copy(
                src_hbm.at[pl.ds(off, PSTAGE)], sref, sem).start()

        def wait_stage(dref, sref, sem):
            pltpu.make_async_copy(
                dst_hbm.at[pl.ds(0, PSTAGE)], dref, sem).wait()
            pltpu.make_async_copy(
                src_hbm.at[pl.ds(0, PSTAGE)], sref, sem).wait()

        def scan_stage(si, dref, sref, carry):
            ebase = si * PSTAGE

            def group(g, carry):
                off0, goff0, off1, goff1 = carry
                dv = dref[pl.ds(g * 16, 16)]
                sv = sref[pl.ds(g * 16, 16)]
                ids = ebase + g * 16 + lane

                m0 = (dv >= base0) & (dv < base0 + RSIZE)
                n0 = jnp.sum(m0.astype(jnp.int32))
                plsc.store_compressed(b0s.at[pl.ds(off0, 16)], sv, mask=m0)
                plsc.store_compressed(b0d.at[pl.ds(off0, 16)], dv, mask=m0)
                plsc.store_compressed(b0i.at[pl.ds(off0, 16)], ids, mask=m0)
                off0 = off0 + n0

                @pl.when(off0 >= FLUSH)
                def _():
                    pltpu.sync_copy(b0s.at[pl.ds(0, FLUSH)],
                                    srcg_hbm.at[pl.ds(pl.multiple_of(r0 * ECAP + goff0, 512), FLUSH)])
                    pltpu.sync_copy(b0d.at[pl.ds(0, FLUSH)],
                                    dstg_hbm.at[pl.ds(pl.multiple_of(r0 * ECAP + goff0, 512), FLUSH)])
                    pltpu.sync_copy(b0i.at[pl.ds(0, FLUSH)],
                                    idsg_hbm.at[pl.ds(pl.multiple_of(r0 * ECAP + goff0, 512), FLUSH)])
                    b0s[pl.ds(0, 16)] = b0s[pl.ds(FLUSH, 16)]
                    b0d[pl.ds(0, 16)] = b0d[pl.ds(FLUSH, 16)]
                    b0i[pl.ds(0, 16)] = b0i[pl.ds(FLUSH, 16)]

                goff0 = goff0 + jnp.where(off0 >= FLUSH, FLUSH, 0)
                off0 = jnp.where(off0 >= FLUSH, off0 - FLUSH, off0)

                m1 = (dv >= base1) & (dv < base1 + RSIZE)
                n1 = jnp.sum(m1.astype(jnp.int32))
                plsc.store_compressed(b1s.at[pl.ds(off1, 16)], sv, mask=m1)
                plsc.store_compressed(b1d.at[pl.ds(off1, 16)], dv, mask=m1)
                plsc.store_compressed(b1i.at[pl.ds(off1, 16)], ids, mask=m1)
                off1 = off1 + n1

                @pl.when(off1 >= FLUSH)
                def _():
                    pltpu.sync_copy(b1s.at[pl.ds(0, FLUSH)],
                                    srcg_hbm.at[pl.ds(pl.multiple_of(r1 * ECAP + goff1, 512), FLUSH)])
                    pltpu.sync_copy(b1d.at[pl.ds(0, FLUSH)],
                                    dstg_hbm.at[pl.ds(pl.multiple_of(r1 * ECAP + goff1, 512), FLUSH)])
                    pltpu.sync_copy(b1i.at[pl.ds(0, FLUSH)],
                                    idsg_hbm.at[pl.ds(pl.multiple_of(r1 * ECAP + goff1, 512), FLUSH)])
                    b1s[pl.ds(0, 16)] = b1s[pl.ds(FLUSH, 16)]
                    b1d[pl.ds(0, 16)] = b1d[pl.ds(FLUSH, 16)]
                    b1i[pl.ds(0, 16)] = b1i[pl.ds(FLUSH, 16)]

                goff1 = goff1 + jnp.where(off1 >= FLUSH, FLUSH, 0)
                off1 = jnp.where(off1 >= FLUSH, off1 - FLUSH, off1)
                return (off0, goff0, off1, goff1)

            return lax.fori_loop(0, PSTAGE // 16, group, carry)

        zero = jnp.int32(0)
        carry = (zero, zero, zero, zero)
        issue_stage(0, dstA, srcA, semA)
        nst = E // PSTAGE

        def pair(sp, carry):
            issue_stage(2 * sp + 1, dstB, srcB, semB)
            wait_stage(dstA, srcA, semA)
            carry = scan_stage(2 * sp, dstA, srcA, carry)
            issue_stage(jnp.minimum(2 * sp + 2, nst - 1), dstA, srcA, semA)
            wait_stage(dstB, srcB, semB)
            carry = scan_stage(2 * sp + 1, dstB, srcB, carry)
            return carry

        carry = lax.fori_loop(0, nst // 2, pair, carry)
        wait_stage(dstA, srcA, semA)
        off0, goff0, off1, goff1 = carry

        pad_dst0 = jnp.broadcast_to(jnp.int32(base0 + TRASH), (16,))
        pad_dst1 = jnp.broadcast_to(jnp.int32(base1 + TRASH), (16,))
        pad_zero = jnp.broadcast_to(jnp.int32(0), (16,))
        for p in range(STAGE // 16):
            b0s[pl.ds(off0 + p * 16, 16)] = pad_zero
            b0d[pl.ds(off0 + p * 16, 16)] = pad_dst0
            b0i[pl.ds(off0 + p * 16, 16)] = pad_zero
            b1s[pl.ds(off1 + p * 16, 16)] = pad_zero
            b1d[pl.ds(off1 + p * 16, 16)] = pad_dst1
            b1i[pl.ds(off1 + p * 16, 16)] = pad_zero
        pltpu.sync_copy(b0s, srcg_hbm.at[pl.ds(pl.multiple_of(r0 * ECAP + goff0, 512), BUF)])
        pltpu.sync_copy(b0d, dstg_hbm.at[pl.ds(pl.multiple_of(r0 * ECAP + goff0, 512), BUF)])
        pltpu.sync_copy(b0i, idsg_hbm.at[pl.ds(pl.multiple_of(r0 * ECAP + goff0, 512), BUF)])
        pltpu.sync_copy(b1s, srcg_hbm.at[pl.ds(pl.multiple_of(r1 * ECAP + goff1, 512), BUF)])
        pltpu.sync_copy(b1d, dstg_hbm.at[pl.ds(pl.multiple_of(r1 * ECAP + goff1, 512), BUF)])
        pltpu.sync_copy(b1i, idsg_hbm.at[pl.ds(pl.multiple_of(r1 * ECAP + goff1, 512), BUF)])

        tot0 = goff0 + off0
        tot1 = goff1 + off1
        padded0 = ((tot0 + STAGE - 1) // STAGE) * STAGE
        padded1 = ((tot1 + STAGE - 1) // STAGE) * STAGE
        cbuf[pl.ds(0, 16)] = jnp.broadcast_to(padded0, (16,))
        pltpu.sync_copy(cbuf, cnt_hbm.at[pl.ds(pl.multiple_of(r0 * 16, 16), 16)])
        cbuf[pl.ds(0, 16)] = jnp.broadcast_to(padded1, (16,))
        pltpu.sync_copy(cbuf, cnt_hbm.at[pl.ds(pl.multiple_of(r1 * 16, 16), 16)])

    return part


def _make_edge_pass():
    mesh = plsc.VectorSubcoreMesh(**_SC_MESH)
    out_type = (
        jax.ShapeDtypeStruct((NPAD, D), jnp.float32),
        jax.ShapeDtypeStruct((NPAD, D), jnp.float32),
        jax.ShapeDtypeStruct((NPAD, D), jnp.float32),
        jax.ShapeDtypeStruct((NPAD, D), jnp.float32),
        jax.ShapeDtypeStruct((NPAD, 16), jnp.float32),
    )
    scratch = [
        pltpu.VMEM((STAGE,), jnp.int32),
        pltpu.VMEM((STAGE,), jnp.int32),
        pltpu.VMEM((STAGE,), jnp.int32),
        pltpu.VMEM((16,), jnp.int32),
        pltpu.VMEM((CHUNK, D), jnp.float32),
        pltpu.VMEM((CHUNK, D), jnp.float32),
        pltpu.VMEM((CHUNK, D), jnp.float32),
        pltpu.VMEM((CHUNK, D), jnp.float32),
        pltpu.VMEM((RSIZE + 1, D), jnp.float32),
        pltpu.VMEM((RSIZE + 1, D), jnp.float32),
        pltpu.VMEM((RSIZE + 1, D), jnp.float32),
        pltpu.VMEM((RSIZE + 1, D), jnp.float32),
        pltpu.VMEM((RSIZE + 1, 16), jnp.float32),
        pltpu.SemaphoreType.DMA,
        pltpu.SemaphoreType.DMA,
    ]

    @functools.partial(
        pl.kernel, out_type=out_type, mesh=mesh,
        compiler_params=pltpu.CompilerParams(needs_layout_passes=False),
        scratch_types=scratch)
    def edge_pass(srcg, dstg, idsg, cntr, B_hbm, Ea_hbm,
                  S1_hbm, S2_hbm, MX_hbm, MN_hbm, CNT_hbm,
                  src_st, dst_st, ids_st, cbuf,
                  bA, eA, bB, eB, aS1, aS2, aMX, aMN, aC, semA, semB):
        wid = lax.axis_index("s") * 2 + lax.axis_index("c")
        zero16 = jnp.zeros((16,), jnp.float32)
        neg16 = jnp.full((16,), -3.0e38, jnp.float32)
        pos16 = jnp.full((16,), 3.0e38, jnp.float32)
        one16 = jnp.ones((16,), jnp.float32)

        for rk in range(2):
            r = wid * 2 + rk
            base = r * RSIZE

            def initrow(i, _):
                for cc in range(D // 16):
                    sl = pl.ds(cc * 16, 16)
                    aS1[i, sl] = zero16
                    aS2[i, sl] = zero16
                    aMX[i, sl] = neg16
                    aMN[i, sl] = pos16
                aC[i, pl.ds(0, 16)] = zero16
                return 0

            lax.fori_loop(0, RSIZE + 1, initrow, 0)

            pltpu.sync_copy(cntr.at[pl.ds(pl.multiple_of(r * 16, 16), 16)], cbuf)
            cval = cbuf[pl.ds(0, 16)][0]
            nstages = cval // STAGE

            def issue_chunk(k, bbuf, ebuf, sem):
                pltpu.make_async_copy(
                    B_hbm.at[src_st.at[pl.ds(k * CHUNK, CHUNK)]],
                    bbuf, sem).start()
                pltpu.make_async_copy(
                    Ea_hbm.at[ids_st.at[pl.ds(k * CHUNK, CHUNK)]],
                    ebuf, sem).start()

            def wait_chunk(bbuf, ebuf, sem):
                pltpu.make_async_copy(
                    B_hbm.at[src_st.at[pl.ds(0, CHUNK)]], bbuf, sem).wait()
                pltpu.make_async_copy(
                    Ea_hbm.at[ids_st.at[pl.ds(0, CHUNK)]], ebuf, sem).wait()

            def compute_chunk(k, bbuf, ebuf):
                koff = k * CHUNK

                def group(gg, _):
                    dl = dst_st[pl.ds(koff + gg * 16, 16)] - base
                    for j in range(16):
                        dj = dl[j]
                        ee = gg * 16 + j
                        aC[dj, pl.ds(0, 16)] = aC[dj, pl.ds(0, 16)] + one16
                        for cc in range(D // 16):
                            sl = pl.ds(cc * 16, 16)
                            q = bbuf[ee, sl] + ebuf[ee, sl]
                            aS1[dj, sl] = aS1[dj, sl] + q
                            aS2[dj, sl] = aS2[dj, sl] + q * q
                            aMX[dj, sl] = jnp.maximum(aMX[dj, sl], q)
                            aMN[dj, sl] = jnp.minimum(aMN[dj, sl], q)
                    return 0

                lax.fori_loop(0, CHUNK // 16, group, 0)

            def stage(si, _):
                soff = si * STAGE
                pltpu.sync_copy(srcg.at[pl.ds(pl.multiple_of(r * ECAP + soff, 512), STAGE)], src_st)
                pltpu.sync_copy(dstg.at[pl.ds(pl.multiple_of(r * ECAP + soff, 512), STAGE)], dst_st)
                pltpu.sync_copy(idsg.at[pl.ds(pl.multiple_of(r * ECAP + soff, 512), STAGE)], ids_st)
                issue_chunk(0, bA, eA, semA)

                def kk_body(kk, _):
                    issue_chunk(2 * kk + 1, bB, eB, semB)
                    wait_chunk(bA, eA, semA)
                    compute_chunk(2 * kk, bA, eA)
                    issue_chunk(jnp.minimum(2 * kk + 2, STAGE // CHUNK - 1),
                                bA, eA, semA)
                    wait_chunk(bB, eB, semB)
                    compute_chunk(2 * kk + 1, bB, eB)
                    return 0

                lax.fori_loop(0, STAGE // CHUNK // 2, kk_body, 0)
                wait_chunk(bA, eA, semA)
                return 0

            lax.fori_loop(0, nstages, stage, 0)

            pltpu.sync_copy(aS1.at[pl.ds(0, RSIZE)],
                            S1_hbm.at[pl.ds(base, RSIZE)])
            pltpu.sync_copy(aS2.at[pl.ds(0, RSIZE)],
                            S2_hbm.at[pl.ds(base, RSIZE)])
            pltpu.sync_copy(aMX.at[pl.ds(0, RSIZE)],
                            MX_hbm.at[pl.ds(base, RSIZE)])
            pltpu.sync_copy(aMN.at[pl.ds(0, RSIZE)],
                            MN_hbm.at[pl.ds(base, RSIZE)])
            pltpu.sync_copy(aC.at[pl.ds(0, RSIZE)],
                            CNT_hbm.at[pl.ds(base, RSIZE)])

    return edge_pass


_BE = 2048


def _encode_body(ea_ref, We1, be1, Wp1, bp1, We2, be2, Wp2, bp2, o1, o2):
    eb = ea_ref[...]
    Wf1 = jnp.dot(We1[...], Wp1[...], preferred_element_type=jnp.float32)
    bf1 = jnp.dot(be1[...], Wp1[...], preferred_element_type=jnp.float32) + bp1[...]
    o1[...] = jnp.dot(eb, Wf1, preferred_element_type=jnp.float32) + bf1
    Wf2 = jnp.dot(We2[...], Wp2[...], preferred_element_type=jnp.float32)
    bf2 = jnp.dot(be2[...], Wp2[...], preferred_element_type=jnp.float32) + bp2[...]
    o2[...] = jnp.dot(eb, Wf2, preferred_element_type=jnp.float32) + bf2


def _encode_call(edge_attr, We1, be1, Wp1, bp1, We2, be2, Wp2, bp2):
    nb = pl.cdiv(E, _BE)
    full = lambda shape: pl.BlockSpec(shape, lambda i: (0, 0))
    return pl.pallas_call(
        _encode_body,
        grid=(nb,),
        in_specs=[
            pl.BlockSpec((_BE, ED), lambda i: (i, 0)),
            full((ED, D)), full((1, D)), full((D, D)), full((1, D)),
            full((ED, D)), full((1, D)), full((D, D)), full((1, D)),
        ],
        out_specs=[pl.BlockSpec((_BE, D), lambda i: (i, 0)),
                   pl.BlockSpec((_BE, D), lambda i: (i, 0))],
        out_shape=[jax.ShapeDtypeStruct((E, D), jnp.float32),
                   jax.ShapeDtypeStruct((E, D), jnp.float32)],
    )(edge_attr, We1, be1, Wp1, bp1, We2, be2, Wp2, bp2)


_BN = 512


def _prep_body(x_ref, Wd, Ws, A_ref, B_ref):
    xb = x_ref[...]
    A_ref[...] = jnp.dot(xb, Wd[...], preferred_element_type=jnp.float32)
    B_ref[...] = jnp.dot(xb, Ws[...], preferred_element_type=jnp.float32)


def _prep_call(x, Wd, Ws):
    nb = pl.cdiv(N, _BN)
    full = lambda shape: pl.BlockSpec(shape, lambda i: (0, 0))
    return pl.pallas_call(
        _prep_body,
        grid=(nb,),
        in_specs=[pl.BlockSpec((_BN, D), lambda i: (i, 0)),
                  full((D, D)), full((D, D))],
        out_specs=[pl.BlockSpec((_BN, D), lambda i: (i, 0)),
                   pl.BlockSpec((_BN, D), lambda i: (i, 0))],
        out_shape=[jax.ShapeDtypeStruct((N, D), jnp.float32),
                   jax.ShapeDtypeStruct((N, D), jnp.float32)],
    )(x, Wd, Ws)


def _make_node_body(with_head):
    def body(x_ref, A_ref, S1_ref, S2_ref, MX_ref, MN_ref, C_ref,
             Wpost, bpost, Wlin, blin, gam, bet, Wh, bh, o_ref):
        xb = x_ref[...]
        Ab = A_ref[...]
        S1 = S1_ref[...]
        S2 = S2_ref[...]
        c = C_ref[...][:, 0:1]
        has = c > 0.0
        denom = jnp.maximum(c, 1.0)
        mean = (c * Ab + S1) / denom
        meansq = (c * Ab * Ab + 2.0 * Ab * S1 + S2) / denom
        std = jnp.sqrt(jax.nn.relu(meansq - mean * mean) + 1e-5)
        mx = jnp.where(has, Ab + MX_ref[...], 0.0)
        mn = jnp.where(has, Ab + MN_ref[...], 0.0)
        agg = jnp.concatenate([mean, mx, mn, std], axis=1)
        logd = jnp.log(denom + 1.0)
        amp = logd * (1.0 / _AVG_LOG)
        att = _AVG_LOG / logd
        h13 = jnp.concatenate([xb, agg, agg * amp, agg * att], axis=1)
        o = jnp.dot(h13, Wpost[...], preferred_element_type=jnp.float32) + bpost[...]
        o = jnp.dot(o, Wlin[...], preferred_element_type=jnp.float32) + blin[...]
        mu = jnp.mean(o, axis=1, keepdims=True)
        var = jnp.mean((o - mu) * (o - mu), axis=1, keepdims=True)
        o = (o - mu) * jax.lax.rsqrt(var + 1e-5) * gam[...] + bet[...]
        o = jax.nn.relu(o)
        if with_head:
            o = jnp.dot(o, Wh[...], preferred_element_type=jnp.float32) + bh[...]
        o_ref[...] = o

    return body


def _node_call(x, A, S1, S2, MX, MN, CNT, Wpost, bpost, Wlin, blin,
               gam, bet, Wh, bh, with_head):
    nb = pl.cdiv(N, _BN)
    odim = Wh.shape[1] if with_head else D
    full = lambda shape: pl.BlockSpec(shape, lambda i: (0, 0))
    return pl.pallas_call(
        _make_node_body(with_head),
        grid=(nb,),
        in_specs=[
            pl.BlockSpec((_BN, D), lambda i: (i, 0)),
            pl.BlockSpec((_BN, D), lambda i: (i, 0)),
            pl.BlockSpec((_BN, D), lambda i: (i, 0)),
            pl.BlockSpec((_BN, D), lambda i: (i, 0)),
            pl.BlockSpec((_BN, D), lambda i: (i, 0)),
            pl.BlockSpec((_BN, D), lambda i: (i, 0)),
            pl.BlockSpec((_BN, 16), lambda i: (i, 0)),
            full((13 * D, D)), full((1, D)), full((D, D)), full((1, D)),
            full((1, D)), full((1, D)), full((D, Wh.shape[1])),
            full((1, Wh.shape[1])),
        ],
        out_specs=pl.BlockSpec((_BN, odim), lambda i: (i, 0)),
        out_shape=jax.ShapeDtypeStruct((N, odim), jnp.float32),
    )(x, A, S1, S2, MX, MN, CNT, Wpost, bpost, Wlin, blin, gam, bet, Wh, bh)


_partition = _make_partition()
_edge_pass = _make_edge_pass()


def kernel(x, edge_index, edge_attr, params):
    src = edge_index[0]
    dst = edge_index[1]
    convs = params["convs"]
    norms = params["norms"]
    Wh, bh = params["head"]
    bh2 = bh.reshape(1, -1)

    p1, p2 = convs
    Ea1, Ea2 = _encode_call(
        edge_attr,
        p1["We"], p1["be"].reshape(1, -1), p1["Wpre"][2 * D:],
        p1["bpre"].reshape(1, -1),
        p2["We"], p2["be"].reshape(1, -1), p2["Wpre"][2 * D:],
        p2["bpre"].reshape(1, -1),
    )
    srcg, dstg, idsg, cntr = _partition(dst, src)

    h = x
    for li, (p, (g, b)) in enumerate(zip(convs, norms)):
        with_head = li == len(convs) - 1
        A, B = _prep_call(h, p["Wpre"][:D], p["Wpre"][D:2 * D])
        Ea = (Ea1, Ea2)[li]
        S1, S2, MX, MN, CNT = _edge_pass(srcg, dstg, idsg, cntr, B, Ea)
        h = _node_call(h, A, S1, S2, MX, MN, CNT,
                       p["Wpost"], p["bpost"].reshape(1, -1),
                       p["Wlin"], p["blin"].reshape(1, -1),
                       g.reshape(1, -1), b.reshape(1, -1),
                       Wh, bh2, with_head)
    return h
